# Initial kernel scaffold; baseline (speedup 1.0000x reference)
import jax
import jax.numpy as jnp
from jax.experimental import pallas as pl


def kernel(x, hyperedge_index, num_nodes, num_edges, w1_0, b1_0, w2_0, b2_0, w1_1, b1_1, w2_1, b2_1, prelu_a):
    raise NotImplementedError("write your pallas kernel here")


# speedup vs baseline: 2.6246x; 2.6246x over previous
import functools

import jax
import jax.numpy as jnp
from jax import lax
from jax.experimental import pallas as pl
from jax.experimental.pallas import tpu as pltpu
from jax.experimental.pallas import tpu_sc as plsc

N_NODES = 10000
N_EDGES = 10000
C = 128
NC = 2
NS = 16
NW = NC * NS
CHUNK = 128
NPAD = 10240
TRASH = 10000
PER_TILE = NPAD // NS
DEGW = 16
IDXW = 8


def _sc_spmm_call(table, gidx, sidx, zeros_rows, cpw):
    mesh = plsc.VectorSubcoreMesh(core_axis_name="c", subcore_axis_name="s")

    w = table.shape[1]
    out_type = [jax.ShapeDtypeStruct((NC, NPAD, w), jnp.float32)]
    scratch = [
        pltpu.VMEM((IDXW, CHUNK), jnp.int32),
        pltpu.VMEM((IDXW, CHUNK), jnp.int32),
        pltpu.VMEM((CHUNK, w), jnp.float32),
        pltpu.VMEM_SHARED((NPAD, w), jnp.float32),
    ]

    def body(table_h, gidx_h, sidx_h, zrows_h, out_h, gi_v, si_v, rows_v,
             acc_sh):
        c = lax.axis_index("c")
        s = lax.axis_index("s")
        wid = s * NC + c
        base = wid * cpw
        row0 = s * PER_TILE
        pltpu.sync_copy(zrows_h, acc_sh.at[pl.ds(row0, PER_TILE)])
        plsc.subcore_barrier()

        def outer(g, carry):
            pltpu.sync_copy(gidx_h.at[pl.ds(base + g * IDXW, IDXW)], gi_v)
            pltpu.sync_copy(sidx_h.at[pl.ds(base + g * IDXW, IDXW)], si_v)

            def step(j, carry2):
                pltpu.sync_copy(table_h.at[gi_v.at[j]], rows_v)
                pltpu.sync_copy(rows_v, acc_sh.at[si_v.at[j]], add=True)
                return carry2

            lax.fori_loop(0, IDXW, step, 0)
            return carry

        lax.fori_loop(0, cpw // IDXW, outer, 0)
        plsc.subcore_barrier()
        pltpu.sync_copy(acc_sh.at[pl.ds(row0, PER_TILE)],
                        out_h.at[c, pl.ds(row0, PER_TILE)])

    fn = pl.kernel(body, out_type=out_type, mesh=mesh, scratch_types=scratch,
                   name="sc_spmm")
    return fn(table, gidx, sidx, zeros_rows)


def _sc_deg_call(nidx, eidx, zeros_rows, ones_rows, cpw):
    mesh = plsc.VectorSubcoreMesh(core_axis_name="c", subcore_axis_name="s")

    out_type = [jax.ShapeDtypeStruct((NC, NPAD, C), jnp.float32),
                jax.ShapeDtypeStruct((NC, NPAD, C), jnp.float32)]
    scratch = [
        pltpu.VMEM((IDXW, CHUNK), jnp.int32),
        pltpu.VMEM((CHUNK, C), jnp.float32),
        pltpu.VMEM_SHARED((NPAD, C), jnp.float32),
    ]

    def body(nidx_h, eidx_h, zrows_h, ones_h, dege_h, degv_h,
             i_v, ones_v, acc_sh):
        c = lax.axis_index("c")
        s = lax.axis_index("s")
        wid = s * NC + c
        base = wid * cpw
        row0 = s * PER_TILE
        pltpu.sync_copy(ones_h, ones_v)

        for idx_h, out_h in ((eidx_h, dege_h), (nidx_h, degv_h)):
            pltpu.sync_copy(zrows_h, acc_sh.at[pl.ds(row0, PER_TILE)])
            plsc.subcore_barrier()

            def outer(g, carry):
                pltpu.sync_copy(idx_h.at[pl.ds(base + g * IDXW, IDXW)], i_v)

                def step(j, carry2):
                    pltpu.sync_copy(ones_v, acc_sh.at[i_v.at[j]], add=True)
                    return carry2

                lax.fori_loop(0, IDXW, step, 0)
                return carry

            lax.fori_loop(0, cpw // IDXW, outer, 0)
            plsc.subcore_barrier()
            pltpu.sync_copy(acc_sh.at[pl.ds(row0, PER_TILE)],
                            out_h.at[c, pl.ds(row0, PER_TILE)])
            plsc.subcore_barrier()

    fn = pl.kernel(body, out_type=out_type, mesh=mesh, scratch_types=scratch,
                   name="sc_deg")
    return fn(nidx, eidx, zeros_rows, ones_rows)


BM = 1024


def _mm(x, w, b):
    def mm_body(x_ref, w_ref, b_ref, o_ref):
        o_ref[...] = (jnp.dot(x_ref[...], w_ref[...],
                              preferred_element_type=jnp.float32)
                      + b_ref[...])

    m = x.shape[0]
    return pl.pallas_call(
        mm_body,
        grid=(m // BM,),
        in_specs=[pl.BlockSpec((BM, C), lambda i: (i, 0)),
                  pl.BlockSpec((C, C), lambda i: (0, 0)),
                  pl.BlockSpec((1, C), lambda i: (0, 0))],
        out_specs=pl.BlockSpec((BM, C), lambda i: (i, 0)),
        out_shape=jax.ShapeDtypeStruct((m, C), jnp.float32),
    )(x, w, b.reshape(1, C))


def _combine(p0, p1, d0, d1, w, b, a11, prelu, mm):
    def comb_body(p0_ref, p1_ref, d0_ref, d1_ref, a_ref, *rest):
        if mm:
            w_ref, b_ref, v_ref, o_ref = rest
        else:
            (v_ref,) = rest
        ssum = p0_ref[...] + p1_ref[...]
        deg = jnp.maximum(d0_ref[...] + d1_ref[...], 1.0)
        v = ssum / deg
        if prelu:
            v = jnp.where(v >= 0, v, a_ref[0, 0] * v)
        v_ref[...] = v
        if mm:
            o_ref[...] = (jnp.dot(v, w_ref[...],
                                  preferred_element_type=jnp.float32)
                          + b_ref[...])

    m = p0.shape[0]
    in_specs = [pl.BlockSpec((BM, C), lambda i: (i, 0)),
                pl.BlockSpec((BM, C), lambda i: (i, 0)),
                pl.BlockSpec((BM, 1), lambda i: (i, 0)),
                pl.BlockSpec((BM, 1), lambda i: (i, 0)),
                pl.BlockSpec((1, 1), lambda i: (0, 0))]
    args = [p0, p1, d0, d1, a11]
    out_shape = [jax.ShapeDtypeStruct((m, C), jnp.float32)]
    out_specs = [pl.BlockSpec((BM, C), lambda i: (i, 0))]
    if mm:
        in_specs += [pl.BlockSpec((C, C), lambda i: (0, 0)),
                     pl.BlockSpec((1, C), lambda i: (0, 0))]
        args += [w, b.reshape(1, C)]
        out_shape += [jax.ShapeDtypeStruct((m, C), jnp.float32)]
        out_specs += [pl.BlockSpec((BM, C), lambda i: (i, 0))]

    res = pl.pallas_call(
        comb_body,
        grid=(m // BM,),
        in_specs=in_specs,
        out_specs=out_specs,
        out_shape=out_shape,
    )(*args)
    return res if mm else res[0]


def kernel(x, hyperedge_index, num_nodes, num_edges,
           w1_0, b1_0, w2_0, b2_0, w1_1, b1_1, w2_1, b2_1, prelu_a):
    n = x.shape[0]
    nnz = hyperedge_index.shape[1]
    cpw = -(-nnz // (NW * CHUNK))
    cpw = -(-cpw // 8) * 8
    nnz_pad = NW * cpw * CHUNK

    node_idx = hyperedge_index[0].astype(jnp.int32)
    edge_idx = hyperedge_index[1].astype(jnp.int32)
    pad = jnp.full((nnz_pad - nnz,), TRASH, jnp.int32)
    nidx = jnp.concatenate([node_idx, pad]).reshape(NW * cpw, CHUNK)
    eidx = jnp.concatenate([edge_idx, pad]).reshape(NW * cpw, CHUNK)

    xp = jnp.concatenate(
        [x.astype(jnp.float32), jnp.zeros((NPAD - n, C), jnp.float32)])
    zeros_rows = jnp.zeros((PER_TILE, C), jnp.float32)
    a11 = prelu_a.reshape(1, 1).astype(jnp.float32)

    spmm = functools.partial(_sc_spmm_call, zeros_rows=zeros_rows, cpw=cpw)

    ones_rows = jnp.ones((CHUNK, C), jnp.float32)
    dep, dvp = _sc_deg_call(nidx, eidx, zeros_rows, ones_rows, cpw)
    dep = dep[:, :, 0:1]
    dvp = dvp[:, :, 0:1]
    de0, de1 = dep[0], dep[1]
    dv0, dv1 = dvp[0], dvp[1]
    x1 = _mm(xp, w1_0, b1_0)
    (ep,) = spmm(x1, nidx, eidx)
    _, m0 = _combine(ep[0], ep[1], de0, de1, w2_0, b2_0, a11,
                     prelu=False, mm=True)
    (xsp,) = spmm(m0, eidx, nidx)
    _, x1l1 = _combine(xsp[0], xsp[1], dv0, dv1, w1_1, b1_1, a11,
                       prelu=True, mm=True)
    (ep1,) = spmm(x1l1, nidx, eidx)
    e1, m1 = _combine(ep1[0], ep1[1], de0, de1, w2_1, b2_1, a11,
                      prelu=False, mm=True)
    (xsp1,) = spmm(m1, eidx, nidx)
    xf = _combine(xsp1[0], xsp1[1], dv0, dv1, None, None, a11,
                  prelu=True, mm=False)
    return (xf[:n], e1[:n])


# speedup vs baseline: 2.7257x; 1.0385x over previous
import functools

import jax
import jax.numpy as jnp
from jax import lax
from jax.experimental import pallas as pl
from jax.experimental.pallas import tpu as pltpu
from jax.experimental.pallas import tpu_sc as plsc

N_NODES = 10000
N_EDGES = 10000
C = 128
NC = 2
NS = 16
NW = NC * NS
CHUNK = 128
NPAD = 10112
TRASH = 10000
PER_TILE = NPAD // NS
IDXW = 4


def _sc_spmm_call(table, gidx, sidx, zeros_rows, cpw):
    mesh = plsc.VectorSubcoreMesh(core_axis_name="c", subcore_axis_name="s")
    wpw = cpw // IDXW

    out_type = [jax.ShapeDtypeStruct((NC, NPAD, C), jnp.float32)]
    scratch = [
        pltpu.VMEM((IDXW, CHUNK), jnp.int32),
        pltpu.VMEM((IDXW, CHUNK), jnp.int32),
        pltpu.VMEM((IDXW, CHUNK), jnp.int32),
        pltpu.VMEM((IDXW, CHUNK), jnp.int32),
        pltpu.VMEM((CHUNK, C), jnp.float32),
        pltpu.VMEM((CHUNK, C), jnp.float32),
        pltpu.VMEM_SHARED((NPAD, C), jnp.float32),
        pltpu.SemaphoreType.DMA,
        pltpu.SemaphoreType.DMA,
        pltpu.SemaphoreType.DMA,
        pltpu.SemaphoreType.DMA,
        pltpu.SemaphoreType.DMA,
        pltpu.SemaphoreType.DMA,
    ]

    def body(table_h, gidx_h, sidx_h, zrows_h, out_h,
             gi0, gi1, si0, si1, r0, r1, acc_sh,
             ga0, ga1, sc0, sc1, ix0, ix1):
        gi = (gi0, gi1)
        si = (si0, si1)
        rows = (r0, r1)
        ga = (ga0, ga1)
        sc = (sc0, sc1)
        ix = (ix0, ix1)
        c = lax.axis_index("c")
        s = lax.axis_index("s")
        wid = s * NC + c
        wbase = wid * wpw
        row0 = s * PER_TILE
        pltpu.sync_copy(zrows_h, acc_sh.at[pl.ds(row0, PER_TILE)])
        plsc.subcore_barrier()

        def idx_load(w, p):
            pltpu.make_async_copy(gidx_h.at[wbase + w], gi[p], ix[p]).start()
            pltpu.make_async_copy(sidx_h.at[wbase + w], si[p], ix[p]).start()

        def idx_wait(p):
            pltpu.make_async_copy(gidx_h.at[wbase], gi[p], ix[p]).wait()
            pltpu.make_async_copy(sidx_h.at[wbase], si[p], ix[p]).wait()

        def gather_start(p, k, b):
            pltpu.make_async_copy(
                table_h.at[gi[p].at[k]], rows[b], ga[b]).start()

        def gather_wait(b):
            pltpu.make_async_copy(
                table_h.at[gi[0].at[0]], rows[b], ga[b]).wait()

        def scatter_start(p, k, b):
            pltpu.make_async_copy(
                rows[b], acc_sh.at[si[p].at[k]], sc[b]).start(add=True)

        def scatter_wait(b):
            pltpu.make_async_copy(
                rows[b], acc_sh.at[si[0].at[0]], sc[b]).wait()

        def emit_window(w, p, first=False, last=False):
            for k in range(IDXW):
                b = k % 2
                gather_wait(b)
                if not (first and k == 0):
                    scatter_wait(1 - b)
                if k == 0 and not first and not last:
                    idx_load(w + 1, 1 - p)
                if k < IDXW - 1:
                    gather_start(p, k + 1, 1 - b)
                elif not last:
                    if not first:
                        idx_wait(1 - p)
                    gather_start(1 - p, 0, 0)
                scatter_start(p, k, b)

        pltpu.sync_copy(gidx_h.at[wbase], gi[0])
        pltpu.sync_copy(sidx_h.at[wbase], si[0])
        pltpu.sync_copy(gidx_h.at[wbase + 1], gi[1])
        pltpu.sync_copy(sidx_h.at[wbase + 1], si[1])
        gather_start(0, 0, 0)
        emit_window(0, 0, first=True)
        emit_window(1, 1)

        def pair(g, carry):
            emit_window(2 * g, 0)
            emit_window(2 * g + 1, 1)
            return carry

        lax.fori_loop(1, wpw // 2 - 1, pair, 0)
        emit_window(wpw - 2, 0)
        emit_window(wpw - 1, 1, last=True)
        scatter_wait(1)

        plsc.subcore_barrier()
        pltpu.sync_copy(acc_sh.at[pl.ds(row0, PER_TILE)],
                        out_h.at[c, pl.ds(row0, PER_TILE)])

    fn = pl.kernel(body, out_type=out_type, mesh=mesh, scratch_types=scratch,
                   name="sc_spmm")
    return fn(table, gidx, sidx, zeros_rows)


def _sc_deg_call(nidx, eidx, zeros_rows, ones_rows, cpw):
    mesh = plsc.VectorSubcoreMesh(core_axis_name="c", subcore_axis_name="s")
    wpw = cpw // IDXW

    out_type = [jax.ShapeDtypeStruct((NC, NPAD, C), jnp.float32),
                jax.ShapeDtypeStruct((NC, NPAD, C), jnp.float32)]
    scratch = [
        pltpu.VMEM((IDXW, CHUNK), jnp.int32),
        pltpu.VMEM((CHUNK, C), jnp.float32),
        pltpu.VMEM_SHARED((NPAD, C), jnp.float32),
    ]

    def body(nidx_h, eidx_h, zrows_h, ones_h, dege_h, degv_h,
             i_v, ones_v, acc_sh):
        c = lax.axis_index("c")
        s = lax.axis_index("s")
        wid = s * NC + c
        wbase = wid * wpw
        row0 = s * PER_TILE
        pltpu.sync_copy(ones_h, ones_v)

        for idx_h, out_h in ((eidx_h, dege_h), (nidx_h, degv_h)):
            pltpu.sync_copy(zrows_h, acc_sh.at[pl.ds(row0, PER_TILE)])
            plsc.subcore_barrier()

            def outer(g, carry):
                pltpu.sync_copy(idx_h.at[wbase + g], i_v)
                for k in range(IDXW):
                    pltpu.sync_copy(ones_v, acc_sh.at[i_v.at[k]], add=True)
                return carry

            lax.fori_loop(0, wpw, outer, 0)
            plsc.subcore_barrier()
            pltpu.sync_copy(acc_sh.at[pl.ds(row0, PER_TILE)],
                            out_h.at[c, pl.ds(row0, PER_TILE)])
            plsc.subcore_barrier()

    fn = pl.kernel(body, out_type=out_type, mesh=mesh, scratch_types=scratch,
                   name="sc_deg")
    return fn(nidx, eidx, zeros_rows, ones_rows)


BM = 1264


def _mm(x, w, b):
    def mm_body(x_ref, w_ref, b_ref, o_ref):
        o_ref[...] = (jnp.dot(x_ref[...], w_ref[...],
                              preferred_element_type=jnp.float32)
                      + b_ref[...])

    m = x.shape[0]
    return pl.pallas_call(
        mm_body,
        grid=(m // BM,),
        in_specs=[pl.BlockSpec((BM, C), lambda i: (i, 0)),
                  pl.BlockSpec((C, C), lambda i: (0, 0)),
                  pl.BlockSpec((1, C), lambda i: (0, 0))],
        out_specs=pl.BlockSpec((BM, C), lambda i: (i, 0)),
        out_shape=jax.ShapeDtypeStruct((m, C), jnp.float32),
    )(x, w, b.reshape(1, C))


def _combine(p0, p1, d0, d1, w, b, a11, prelu, mm):
    def comb_body(p0_ref, p1_ref, d0_ref, d1_ref, a_ref, *rest):
        if mm:
            w_ref, b_ref, v_ref, o_ref = rest
        else:
            (v_ref,) = rest
        ssum = p0_ref[...] + p1_ref[...]
        deg = jnp.maximum(d0_ref[...] + d1_ref[...], 1.0)
        v = ssum / deg
        if prelu:
            v = jnp.where(v >= 0, v, a_ref[0, 0] * v)
        v_ref[...] = v
        if mm:
            o_ref[...] = (jnp.dot(v, w_ref[...],
                                  preferred_element_type=jnp.float32)
                          + b_ref[...])

    m = p0.shape[0]
    in_specs = [pl.BlockSpec((BM, C), lambda i: (i, 0)),
                pl.BlockSpec((BM, C), lambda i: (i, 0)),
                pl.BlockSpec((BM, 1), lambda i: (i, 0)),
                pl.BlockSpec((BM, 1), lambda i: (i, 0)),
                pl.BlockSpec((1, 1), lambda i: (0, 0))]
    args = [p0, p1, d0, d1, a11]
    out_shape = [jax.ShapeDtypeStruct((m, C), jnp.float32)]
    out_specs = [pl.BlockSpec((BM, C), lambda i: (i, 0))]
    if mm:
        in_specs += [pl.BlockSpec((C, C), lambda i: (0, 0)),
                     pl.BlockSpec((1, C), lambda i: (0, 0))]
        args += [w, b.reshape(1, C)]
        out_shape += [jax.ShapeDtypeStruct((m, C), jnp.float32)]
        out_specs += [pl.BlockSpec((BM, C), lambda i: (i, 0))]

    res = pl.pallas_call(
        comb_body,
        grid=(m // BM,),
        in_specs=in_specs,
        out_specs=out_specs,
        out_shape=out_shape,
    )(*args)
    return res if mm else res[0]


def kernel(x, hyperedge_index, num_nodes, num_edges,
           w1_0, b1_0, w2_0, b2_0, w1_1, b1_1, w2_1, b2_1, prelu_a):
    n = x.shape[0]
    nnz = hyperedge_index.shape[1]
    cpw = -(-nnz // (NW * CHUNK))
    cpw = -(-cpw // (2 * IDXW)) * (2 * IDXW)
    nnz_pad = NW * cpw * CHUNK

    node_idx = hyperedge_index[0].astype(jnp.int32)
    edge_idx = hyperedge_index[1].astype(jnp.int32)
    pad = jnp.full((nnz_pad - nnz,), TRASH, jnp.int32)
    nidx = jnp.concatenate([node_idx, pad]).reshape(-1, IDXW, CHUNK)
    eidx = jnp.concatenate([edge_idx, pad]).reshape(-1, IDXW, CHUNK)

    xp = jnp.concatenate(
        [x.astype(jnp.float32), jnp.zeros((NPAD - n, C), jnp.float32)])
    zeros_rows = jnp.zeros((PER_TILE, C), jnp.float32)
    a11 = prelu_a.reshape(1, 1).astype(jnp.float32)

    spmm = functools.partial(_sc_spmm_call, zeros_rows=zeros_rows, cpw=cpw)

    ones_rows = jnp.ones((CHUNK, C), jnp.float32)
    dep, dvp = _sc_deg_call(nidx, eidx, zeros_rows, ones_rows, cpw)
    dep = dep[:, :, 0:1]
    dvp = dvp[:, :, 0:1]
    de0, de1 = dep[0], dep[1]
    dv0, dv1 = dvp[0], dvp[1]
    x1 = _mm(xp, w1_0, b1_0)
    (ep,) = spmm(x1, nidx, eidx)
    _, m0 = _combine(ep[0], ep[1], de0, de1, w2_0, b2_0, a11,
                     prelu=False, mm=True)
    (xsp,) = spmm(m0, eidx, nidx)
    _, x1l1 = _combine(xsp[0], xsp[1], dv0, dv1, w1_1, b1_1, a11,
                       prelu=True, mm=True)
    (ep1,) = spmm(x1l1, nidx, eidx)
    e1, m1 = _combine(ep1[0], ep1[1], de0, de1, w2_1, b2_1, a11,
                      prelu=False, mm=True)
    (xsp1,) = spmm(m1, eidx, nidx)
    xf = _combine(xsp1[0], xsp1[1], dv0, dv1, None, None, a11,
                  prelu=True, mm=False)
    return (xf[:n], e1[:n])


# speedup vs baseline: 2.8919x; 1.0610x over previous
import functools

import jax
import jax.numpy as jnp
from jax import lax
from jax.experimental import pallas as pl
from jax.experimental.pallas import tpu as pltpu
from jax.experimental.pallas import tpu_sc as plsc

N_NODES = 10000
N_EDGES = 10000
C = 128
NC = 2
NS = 16
NW = NC * NS
CHUNK = 128
NPAD = 10112
TRASH = 10000
PER_TILE = NPAD // NS
IDXW = 4


def _sc_spmm_call(table, gidx, sidx, zeros_rows, cpw):
    mesh = plsc.VectorSubcoreMesh(core_axis_name="c", subcore_axis_name="s")
    wpw = cpw // IDXW

    out_type = [jax.ShapeDtypeStruct((NC, NPAD, C), jnp.float32)]
    scratch = [
        pltpu.VMEM((IDXW, CHUNK), jnp.int32),
        pltpu.VMEM((IDXW, CHUNK), jnp.int32),
        pltpu.VMEM((IDXW, CHUNK), jnp.int32),
        pltpu.VMEM((IDXW, CHUNK), jnp.int32),
        pltpu.VMEM((CHUNK, C), jnp.float32),
        pltpu.VMEM((CHUNK, C), jnp.float32),
        pltpu.VMEM_SHARED((NPAD, C), jnp.float32),
        pltpu.SemaphoreType.DMA,
        pltpu.SemaphoreType.DMA,
        pltpu.SemaphoreType.DMA,
        pltpu.SemaphoreType.DMA,
        pltpu.SemaphoreType.DMA,
        pltpu.SemaphoreType.DMA,
    ]

    def body(table_h, gidx_h, sidx_h, zrows_h, out_h,
             gi0, gi1, si0, si1, r0, r1, acc_sh,
             ga0, ga1, sc0, sc1, ix0, ix1):
        c = lax.axis_index("c")
        s = lax.axis_index("s")
        wid = s * NC + c
        row0 = s * PER_TILE
        pltpu.sync_copy(zrows_h, acc_sh.at[pl.ds(row0, PER_TILE)])
        plsc.subcore_barrier()
        _emit_spmm_pipeline(table_h, gidx_h, sidx_h, acc_sh,
                            (gi0, gi1), (si0, si1), (r0, r1),
                            (ga0, ga1), (sc0, sc1), (ix0, ix1),
                            wid * wpw, wpw)
        plsc.subcore_barrier()
        pltpu.sync_copy(acc_sh.at[pl.ds(row0, PER_TILE)],
                        out_h.at[c, pl.ds(row0, PER_TILE)])

    fn = pl.kernel(body, out_type=out_type, mesh=mesh, scratch_types=scratch,
                   name="sc_spmm")
    return fn(table, gidx, sidx, zeros_rows)


def _emit_spmm_pipeline(table_h, gidx_h, sidx_h, acc_sh, gi, si, rows,
                        ga, sc, ix, wbase, wpw):
    def idx_load(w, p):
        pltpu.make_async_copy(gidx_h.at[wbase + w], gi[p], ix[p]).start()
        pltpu.make_async_copy(sidx_h.at[wbase + w], si[p], ix[p]).start()

    def idx_wait(p):
        pltpu.make_async_copy(gidx_h.at[wbase], gi[p], ix[p]).wait()
        pltpu.make_async_copy(sidx_h.at[wbase], si[p], ix[p]).wait()

    def gather_start(p, k, b):
        pltpu.make_async_copy(
            table_h.at[gi[p].at[k]], rows[b], ga[b]).start()

    def gather_wait(b):
        pltpu.make_async_copy(
            table_h.at[gi[0].at[0]], rows[b], ga[b]).wait()

    def scatter_start(p, k, b):
        pltpu.make_async_copy(
            rows[b], acc_sh.at[si[p].at[k]], sc[b]).start(add=True)

    def scatter_wait(b):
        pltpu.make_async_copy(
            rows[b], acc_sh.at[si[0].at[0]], sc[b]).wait()

    def emit_window(w, p, first=False, last=False):
        for k in range(IDXW):
            b = k % 2
            gather_wait(b)
            if not (first and k == 0):
                scatter_wait(1 - b)
            if k == 0 and not first and not last:
                idx_load(w + 1, 1 - p)
            if k < IDXW - 1:
                gather_start(p, k + 1, 1 - b)
            elif not last:
                if not first:
                    idx_wait(1 - p)
                gather_start(1 - p, 0, 0)
            scatter_start(p, k, b)

    pltpu.sync_copy(gidx_h.at[wbase], gi[0])
    pltpu.sync_copy(sidx_h.at[wbase], si[0])
    pltpu.sync_copy(gidx_h.at[wbase + 1], gi[1])
    pltpu.sync_copy(sidx_h.at[wbase + 1], si[1])
    gather_start(0, 0, 0)
    emit_window(0, 0, first=True)
    emit_window(1, 1)

    def pair(g, carry):
        emit_window(2 * g, 0)
        emit_window(2 * g + 1, 1)
        return carry

    lax.fori_loop(1, wpw // 2 - 1, pair, 0)
    emit_window(wpw - 2, 0)
    emit_window(wpw - 1, 1, last=True)
    scatter_wait(1)


def _sc_spmm_deg_call(table, gidx, sidx, zeros_rows, ones_rows, cpw):
    mesh = plsc.VectorSubcoreMesh(core_axis_name="c", subcore_axis_name="s")
    wpw2 = 2 * (cpw // IDXW)

    out_type = [jax.ShapeDtypeStruct((NPAD, C), jnp.float32),
                jax.ShapeDtypeStruct((NPAD, C), jnp.float32),
                jax.ShapeDtypeStruct((NPAD, C), jnp.float32)]
    scratch = [
        pltpu.VMEM((IDXW, CHUNK), jnp.int32),
        pltpu.VMEM((IDXW, CHUNK), jnp.int32),
        pltpu.VMEM((IDXW, CHUNK), jnp.int32),
        pltpu.VMEM((IDXW, CHUNK), jnp.int32),
        pltpu.VMEM((CHUNK, C), jnp.float32),
        pltpu.VMEM((CHUNK, C), jnp.float32),
        pltpu.VMEM_SHARED((NPAD, C), jnp.float32),
        pltpu.SemaphoreType.DMA,
        pltpu.SemaphoreType.DMA,
        pltpu.SemaphoreType.DMA,
        pltpu.SemaphoreType.DMA,
        pltpu.SemaphoreType.DMA,
        pltpu.SemaphoreType.DMA,
    ]

    def body(table_h, gidx_h, sidx_h, zrows_h, ones_h,
             out_h, dege_h, degv_h,
             gi0, gi1, si0, si1, r0, r1, acc_sh,
             ga0, ga1, sc0, sc1, ix0, ix1):
        c = lax.axis_index("c")
        s = lax.axis_index("s")
        wbase = s * wpw2
        row0 = s * PER_TILE

        @pl.when(c == 0)
        def _spmm():
            pltpu.sync_copy(zrows_h, acc_sh.at[pl.ds(row0, PER_TILE)])
            plsc.subcore_barrier()
            _emit_spmm_pipeline(table_h, gidx_h, sidx_h, acc_sh,
                                (gi0, gi1), (si0, si1), (r0, r1),
                                (ga0, ga1), (sc0, sc1), (ix0, ix1),
                                wbase, wpw2)
            plsc.subcore_barrier()
            pltpu.sync_copy(acc_sh.at[pl.ds(row0, PER_TILE)],
                            out_h.at[pl.ds(row0, PER_TILE)])

        @pl.when(c == 1)
        def _deg():
            pltpu.sync_copy(ones_h, r0)
            for idx_h, i_v, dout_h in ((sidx_h, si0, dege_h),
                                       (gidx_h, gi0, degv_h)):
                pltpu.sync_copy(zrows_h, acc_sh.at[pl.ds(row0, PER_TILE)])
                plsc.subcore_barrier()

                def outer(g, carry):
                    pltpu.sync_copy(idx_h.at[wbase + g], i_v)
                    for k in range(IDXW):
                        pltpu.sync_copy(r0, acc_sh.at[i_v.at[k]], add=True)
                    return carry

                lax.fori_loop(0, wpw2, outer, 0)
                plsc.subcore_barrier()
                pltpu.sync_copy(acc_sh.at[pl.ds(row0, PER_TILE)],
                                dout_h.at[pl.ds(row0, PER_TILE)])
                plsc.subcore_barrier()

    fn = pl.kernel(body, out_type=out_type, mesh=mesh, scratch_types=scratch,
                   name="sc_spmm_deg")
    return fn(table, gidx, sidx, zeros_rows, ones_rows)


BM = 1264


def _mm(x, w, b):
    def mm_body(x_ref, w_ref, b_ref, o_ref):
        o_ref[...] = (jnp.dot(x_ref[...], w_ref[...],
                              preferred_element_type=jnp.float32)
                      + b_ref[...])

    m = x.shape[0]
    return pl.pallas_call(
        mm_body,
        grid=(m // BM,),
        in_specs=[pl.BlockSpec((BM, C), lambda i: (i, 0)),
                  pl.BlockSpec((C, C), lambda i: (0, 0)),
                  pl.BlockSpec((1, C), lambda i: (0, 0))],
        out_specs=pl.BlockSpec((BM, C), lambda i: (i, 0)),
        out_shape=jax.ShapeDtypeStruct((m, C), jnp.float32),
    )(x, w, b.reshape(1, C))


def _combine(p0, p1, d, w, b, a11, prelu, mm):
    def comb_body(*refs):
        i = 0
        p0_ref = refs[i]; i += 1
        if p1 is not None:
            p1_ref = refs[i]; i += 1
        d_ref = refs[i]; i += 1
        a_ref = refs[i]; i += 1
        if mm:
            w_ref = refs[i]; b_ref = refs[i + 1]
            v_ref = refs[i + 2]; o_ref = refs[i + 3]
        else:
            v_ref = refs[i]
        ssum = p0_ref[...]
        if p1 is not None:
            ssum = ssum + p1_ref[...]
        deg = jnp.maximum(d_ref[...], 1.0)
        v = ssum / deg
        if prelu:
            v = jnp.where(v >= 0, v, a_ref[0, 0] * v)
        v_ref[...] = v
        if mm:
            o_ref[...] = (jnp.dot(v, w_ref[...],
                                  preferred_element_type=jnp.float32)
                          + b_ref[...])

    m = p0.shape[0]
    in_specs = [pl.BlockSpec((BM, C), lambda i: (i, 0))]
    args = [p0]
    if p1 is not None:
        in_specs.append(pl.BlockSpec((BM, C), lambda i: (i, 0)))
        args.append(p1)
    in_specs += [pl.BlockSpec((BM, 1), lambda i: (i, 0)),
                 pl.BlockSpec((1, 1), lambda i: (0, 0))]
    args += [d, a11]
    out_shape = [jax.ShapeDtypeStruct((m, C), jnp.float32)]
    out_specs = [pl.BlockSpec((BM, C), lambda i: (i, 0))]
    if mm:
        in_specs += [pl.BlockSpec((C, C), lambda i: (0, 0)),
                     pl.BlockSpec((1, C), lambda i: (0, 0))]
        args += [w, b.reshape(1, C)]
        out_shape += [jax.ShapeDtypeStruct((m, C), jnp.float32)]
        out_specs += [pl.BlockSpec((BM, C), lambda i: (i, 0))]

    res = pl.pallas_call(
        comb_body,
        grid=(m // BM,),
        in_specs=in_specs,
        out_specs=out_specs,
        out_shape=out_shape,
    )(*args)
    return res if mm else res[0]


def kernel(x, hyperedge_index, num_nodes, num_edges,
           w1_0, b1_0, w2_0, b2_0, w1_1, b1_1, w2_1, b2_1, prelu_a):
    n = x.shape[0]
    nnz = hyperedge_index.shape[1]
    cpw = -(-nnz // (NW * CHUNK))
    cpw = -(-cpw // (2 * IDXW)) * (2 * IDXW)
    nnz_pad = NW * cpw * CHUNK

    node_idx = hyperedge_index[0].astype(jnp.int32)
    edge_idx = hyperedge_index[1].astype(jnp.int32)
    pad = jnp.full((nnz_pad - nnz,), TRASH, jnp.int32)
    nidx = jnp.concatenate([node_idx, pad]).reshape(-1, IDXW, CHUNK)
    eidx = jnp.concatenate([edge_idx, pad]).reshape(-1, IDXW, CHUNK)

    xp = jnp.concatenate(
        [x.astype(jnp.float32), jnp.zeros((NPAD - n, C), jnp.float32)])
    zeros_rows = jnp.zeros((PER_TILE, C), jnp.float32)
    a11 = prelu_a.reshape(1, 1).astype(jnp.float32)

    spmm = functools.partial(_sc_spmm_call, zeros_rows=zeros_rows, cpw=cpw)

    ones_rows = jnp.ones((CHUNK, C), jnp.float32)
    x1 = _mm(xp, w1_0, b1_0)
    ep, dege, degv = _sc_spmm_deg_call(x1, nidx, eidx, zeros_rows,
                                       ones_rows, cpw)
    de = dege[:, 0:1]
    dv = degv[:, 0:1]
    _, m0 = _combine(ep, None, de, w2_0, b2_0, a11, prelu=False, mm=True)
    (xsp,) = spmm(m0, eidx, nidx)
    _, x1l1 = _combine(xsp[0], xsp[1], dv, w1_1, b1_1, a11,
                       prelu=True, mm=True)
    (ep1,) = spmm(x1l1, nidx, eidx)
    e1, m1 = _combine(ep1[0], ep1[1], de, w2_1, b2_1, a11,
                      prelu=False, mm=True)
    (xsp1,) = spmm(m1, eidx, nidx)
    xf = _combine(xsp1[0], xsp1[1], dv, None, None, a11,
                  prelu=True, mm=False)
    return (xf[:n], e1[:n])


# speedup vs baseline: 3.3015x; 1.1416x over previous
import functools

import jax
import jax.numpy as jnp
from jax import lax
from jax.experimental import pallas as pl
from jax.experimental.pallas import tpu as pltpu
from jax.experimental.pallas import tpu_sc as plsc

N_NODES = 10000
N_EDGES = 10000
C = 128
NC = 2
NS = 16
NW = NC * NS
CHUNK = 128
NPAD = 10112
TRASH = 10000
PER_TILE = NPAD // NS
IDXW = 4


WB = 4


def _sc_spmm_call(table, gidx, sidx, zeros_rows, cpw):
    mesh = plsc.VectorSubcoreMesh(core_axis_name="c", subcore_axis_name="s")
    wa = 2 * (cpw // IDXW) - WB

    out_type = [jax.ShapeDtypeStruct((NC, NPAD, C), jnp.float32)]
    scratch = [
        pltpu.VMEM((IDXW, CHUNK), jnp.int32),
        pltpu.VMEM((IDXW, CHUNK), jnp.int32),
        pltpu.VMEM((IDXW, CHUNK), jnp.int32),
        pltpu.VMEM((IDXW, CHUNK), jnp.int32),
        pltpu.VMEM((CHUNK, C), jnp.float32),
        pltpu.VMEM((CHUNK, C), jnp.float32),
        pltpu.VMEM_SHARED((NPAD, C), jnp.float32),
        pltpu.SemaphoreType.DMA,
        pltpu.SemaphoreType.DMA,
        pltpu.SemaphoreType.DMA,
        pltpu.SemaphoreType.DMA,
        pltpu.SemaphoreType.DMA,
        pltpu.SemaphoreType.DMA,
    ]

    def body(table_h, gidx_h, sidx_h, zrows_h, out_h,
             gi0, gi1, si0, si1, r0, r1, acc_sh,
             ga0, ga1, sc0, sc1, ix0, ix1):
        c = lax.axis_index("c")
        s = lax.axis_index("s")
        row0 = s * PER_TILE
        pltpu.sync_copy(zrows_h, acc_sh.at[pl.ds(row0, PER_TILE)])
        plsc.subcore_barrier()
        bufs = ((gi0, gi1), (si0, si1), (r0, r1),
                (ga0, ga1), (sc0, sc1), (ix0, ix1))

        @pl.when(c == 0)
        def _():
            _emit_spmm_pipeline(table_h, gidx_h, sidx_h, acc_sh, *bufs,
                                s * wa, wa)

        @pl.when(c == 1)
        def _():
            _emit_spmm_pipeline(table_h, gidx_h, sidx_h, acc_sh, *bufs,
                                NS * wa + s * WB, WB)

        plsc.subcore_barrier()
        pltpu.sync_copy(acc_sh.at[pl.ds(row0, PER_TILE)],
                        out_h.at[c, pl.ds(row0, PER_TILE)])

    fn = pl.kernel(body, out_type=out_type, mesh=mesh, scratch_types=scratch,
                   name="sc_spmm")
    return fn(table, gidx, sidx, zeros_rows)


def _emit_spmm_pipeline(table_h, gidx_h, sidx_h, acc_sh, gi, si, rows,
                        ga, sc, ix, wbase, wpw):
    def idx_load(w, p):
        pltpu.make_async_copy(gidx_h.at[wbase + w], gi[p], ix[p]).start()
        pltpu.make_async_copy(sidx_h.at[wbase + w], si[p], ix[p]).start()

    def idx_wait(p):
        pltpu.make_async_copy(gidx_h.at[wbase], gi[p], ix[p]).wait()
        pltpu.make_async_copy(sidx_h.at[wbase], si[p], ix[p]).wait()

    def gather_start(p, k, b):
        pltpu.make_async_copy(
            table_h.at[gi[p].at[k]], rows[b], ga[b]).start()

    def gather_wait(b):
        pltpu.make_async_copy(
            table_h.at[gi[0].at[0]], rows[b], ga[b]).wait()

    def scatter_start(p, k, b):
        pltpu.make_async_copy(
            rows[b], acc_sh.at[si[p].at[k]], sc[b]).start(add=True)

    def scatter_wait(b):
        pltpu.make_async_copy(
            rows[b], acc_sh.at[si[0].at[0]], sc[b]).wait()

    def emit_window(w, p, first=False, last=False):
        for k in range(IDXW):
            b = k % 2
            gather_wait(b)
            if not (first and k == 0):
                scatter_wait(1 - b)
            if k == 0 and not first and not last:
                idx_load(w + 1, 1 - p)
            if k < IDXW - 1:
                gather_start(p, k + 1, 1 - b)
            elif not last:
                if not first:
                    idx_wait(1 - p)
                gather_start(1 - p, 0, 0)
            scatter_start(p, k, b)

    pltpu.sync_copy(gidx_h.at[wbase], gi[0])
    pltpu.sync_copy(sidx_h.at[wbase], si[0])
    pltpu.sync_copy(gidx_h.at[wbase + 1], gi[1])
    pltpu.sync_copy(sidx_h.at[wbase + 1], si[1])
    gather_start(0, 0, 0)
    emit_window(0, 0, first=True)
    emit_window(1, 1)

    def pair(g, carry):
        emit_window(2 * g, 0)
        emit_window(2 * g + 1, 1)
        return carry

    lax.fori_loop(1, wpw // 2 - 1, pair, 0)
    emit_window(wpw - 2, 0)
    emit_window(wpw - 1, 1, last=True)
    scatter_wait(1)


def _sc_spmm_deg_call(table, gidx, sidx, zeros_rows, ones_rows, cpw):
    mesh = plsc.VectorSubcoreMesh(core_axis_name="c", subcore_axis_name="s")
    wa = 2 * (cpw // IDXW) - WB
    wpw2 = 2 * (cpw // IDXW)

    out_type = [jax.ShapeDtypeStruct((NC, NPAD, C), jnp.float32),
                jax.ShapeDtypeStruct((NPAD, C), jnp.float32),
                jax.ShapeDtypeStruct((NPAD, C), jnp.float32)]
    scratch = [
        pltpu.VMEM((IDXW, CHUNK), jnp.int32),
        pltpu.VMEM((IDXW, CHUNK), jnp.int32),
        pltpu.VMEM((IDXW, CHUNK), jnp.int32),
        pltpu.VMEM((IDXW, CHUNK), jnp.int32),
        pltpu.VMEM((CHUNK, C), jnp.float32),
        pltpu.VMEM((CHUNK, C), jnp.float32),
        pltpu.VMEM_SHARED((NPAD, C), jnp.float32),
        pltpu.SemaphoreType.DMA,
        pltpu.SemaphoreType.DMA,
        pltpu.SemaphoreType.DMA,
        pltpu.SemaphoreType.DMA,
        pltpu.SemaphoreType.DMA,
        pltpu.SemaphoreType.DMA,
    ]

    def body(table_h, gidx_h, sidx_h, zrows_h, ones_h,
             out_h, dege_h, degv_h,
             gi0, gi1, si0, si1, r0, r1, acc_sh,
             ga0, ga1, sc0, sc1, ix0, ix1):
        c = lax.axis_index("c")
        s = lax.axis_index("s")
        wbase = s * wpw2
        row0 = s * PER_TILE
        bufs = ((gi0, gi1), (si0, si1), (r0, r1),
                (ga0, ga1), (sc0, sc1), (ix0, ix1))

        @pl.when(c == 0)
        def _spmm():
            pltpu.sync_copy(zrows_h, acc_sh.at[pl.ds(row0, PER_TILE)])
            plsc.subcore_barrier()
            _emit_spmm_pipeline(table_h, gidx_h, sidx_h, acc_sh, *bufs,
                                s * wa, wa)
            plsc.subcore_barrier()
            pltpu.sync_copy(acc_sh.at[pl.ds(row0, PER_TILE)],
                            out_h.at[0, pl.ds(row0, PER_TILE)])

        @pl.when(c == 1)
        def _deg():
            pltpu.sync_copy(ones_h, r0)
            for idx_h, i_v, dout_h in ((sidx_h, si0, dege_h),
                                       (gidx_h, gi0, degv_h)):
                pltpu.sync_copy(zrows_h, acc_sh.at[pl.ds(row0, PER_TILE)])
                plsc.subcore_barrier()

                def outer(g, carry):
                    pltpu.sync_copy(idx_h.at[wbase + g], i_v)
                    for k in range(IDXW):
                        pltpu.sync_copy(r0, acc_sh.at[i_v.at[k]], add=True)
                    return carry

                lax.fori_loop(0, wpw2, outer, 0)
                plsc.subcore_barrier()
                pltpu.sync_copy(acc_sh.at[pl.ds(row0, PER_TILE)],
                                dout_h.at[pl.ds(row0, PER_TILE)])
                plsc.subcore_barrier()
            pltpu.sync_copy(zrows_h, acc_sh.at[pl.ds(row0, PER_TILE)])
            plsc.subcore_barrier()
            _emit_spmm_pipeline(table_h, gidx_h, sidx_h, acc_sh, *bufs,
                                NS * wa + s * WB, WB)
            plsc.subcore_barrier()
            pltpu.sync_copy(acc_sh.at[pl.ds(row0, PER_TILE)],
                            out_h.at[1, pl.ds(row0, PER_TILE)])

    fn = pl.kernel(body, out_type=out_type, mesh=mesh, scratch_types=scratch,
                   name="sc_spmm_deg")
    return fn(table, gidx, sidx, zeros_rows, ones_rows)


BM = 1264


def _mm(x, w, b):
    def mm_body(x_ref, w_ref, b_ref, o_ref):
        o_ref[...] = (jnp.dot(x_ref[...], w_ref[...],
                              preferred_element_type=jnp.float32)
                      + b_ref[...])

    m = x.shape[0]
    return pl.pallas_call(
        mm_body,
        grid=(m // BM,),
        in_specs=[pl.BlockSpec((BM, C), lambda i: (i, 0)),
                  pl.BlockSpec((C, C), lambda i: (0, 0)),
                  pl.BlockSpec((1, C), lambda i: (0, 0))],
        out_specs=pl.BlockSpec((BM, C), lambda i: (i, 0)),
        out_shape=jax.ShapeDtypeStruct((m, C), jnp.float32),
    )(x, w, b.reshape(1, C))


def _combine(p0, p1, d, w, b, a11, prelu, mm):
    def comb_body(*refs):
        i = 0
        p0_ref = refs[i]; i += 1
        if p1 is not None:
            p1_ref = refs[i]; i += 1
        d_ref = refs[i]; i += 1
        a_ref = refs[i]; i += 1
        if mm:
            w_ref = refs[i]; b_ref = refs[i + 1]
            v_ref = refs[i + 2]; o_ref = refs[i + 3]
        else:
            v_ref = refs[i]
        ssum = p0_ref[...]
        if p1 is not None:
            ssum = ssum + p1_ref[...]
        deg = jnp.maximum(d_ref[...], 1.0)
        v = ssum / deg
        if prelu:
            v = jnp.where(v >= 0, v, a_ref[0, 0] * v)
        v_ref[...] = v
        if mm:
            o_ref[...] = (jnp.dot(v, w_ref[...],
                                  preferred_element_type=jnp.float32)
                          + b_ref[...])

    m = p0.shape[0]
    in_specs = [pl.BlockSpec((BM, C), lambda i: (i, 0))]
    args = [p0]
    if p1 is not None:
        in_specs.append(pl.BlockSpec((BM, C), lambda i: (i, 0)))
        args.append(p1)
    in_specs += [pl.BlockSpec((BM, 1), lambda i: (i, 0)),
                 pl.BlockSpec((1, 1), lambda i: (0, 0))]
    args += [d, a11]
    out_shape = [jax.ShapeDtypeStruct((m, C), jnp.float32)]
    out_specs = [pl.BlockSpec((BM, C), lambda i: (i, 0))]
    if mm:
        in_specs += [pl.BlockSpec((C, C), lambda i: (0, 0)),
                     pl.BlockSpec((1, C), lambda i: (0, 0))]
        args += [w, b.reshape(1, C)]
        out_shape += [jax.ShapeDtypeStruct((m, C), jnp.float32)]
        out_specs += [pl.BlockSpec((BM, C), lambda i: (i, 0))]

    res = pl.pallas_call(
        comb_body,
        grid=(m // BM,),
        in_specs=in_specs,
        out_specs=out_specs,
        out_shape=out_shape,
    )(*args)
    return res if mm else res[0]


def kernel(x, hyperedge_index, num_nodes, num_edges,
           w1_0, b1_0, w2_0, b2_0, w1_1, b1_1, w2_1, b2_1, prelu_a):
    n = x.shape[0]
    nnz = hyperedge_index.shape[1]
    cpw = -(-nnz // (NW * CHUNK))
    cpw = -(-cpw // (2 * IDXW)) * (2 * IDXW)
    nnz_pad = NW * cpw * CHUNK

    node_idx = hyperedge_index[0].astype(jnp.int32)
    edge_idx = hyperedge_index[1].astype(jnp.int32)
    pad = jnp.full((nnz_pad - nnz,), TRASH, jnp.int32)
    nidx = jnp.concatenate([node_idx, pad]).reshape(-1, IDXW, CHUNK)
    eidx = jnp.concatenate([edge_idx, pad]).reshape(-1, IDXW, CHUNK)

    xp = jnp.concatenate(
        [x.astype(jnp.float32), jnp.zeros((NPAD - n, C), jnp.float32)])
    zeros_rows = jnp.zeros((PER_TILE, C), jnp.float32)
    a11 = prelu_a.reshape(1, 1).astype(jnp.float32)

    spmm = functools.partial(_sc_spmm_call, zeros_rows=zeros_rows, cpw=cpw)

    ones_rows = jnp.ones((CHUNK, C), jnp.float32)
    x1 = _mm(xp, w1_0, b1_0)
    ep, dege, degv = _sc_spmm_deg_call(x1, nidx, eidx, zeros_rows,
                                       ones_rows, cpw)
    de = dege[:, 0:1]
    dv = degv[:, 0:1]
    _, m0 = _combine(ep[0], ep[1], de, w2_0, b2_0, a11, prelu=False, mm=True)
    (xsp,) = spmm(m0, eidx, nidx)
    _, x1l1 = _combine(xsp[0], xsp[1], dv, w1_1, b1_1, a11,
                       prelu=True, mm=True)
    (ep1,) = spmm(x1l1, nidx, eidx)
    e1, m1 = _combine(ep1[0], ep1[1], de, w2_1, b2_1, a11,
                      prelu=False, mm=True)
    (xsp1,) = spmm(m1, eidx, nidx)
    xf = _combine(xsp1[0], xsp1[1], dv, None, None, a11,
                  prelu=True, mm=False)
    return (xf[:n], e1[:n])


# speedup vs baseline: 3.5064x; 1.0621x over previous
import functools

import jax
import jax.numpy as jnp
from jax import lax
from jax.experimental import pallas as pl
from jax.experimental.pallas import tpu as pltpu
from jax.experimental.pallas import tpu_sc as plsc

N_NODES = 10000
N_EDGES = 10000
C = 128
NC = 2
NS = 16
NW = NC * NS
CHUNK = 128
NPAD = 10112
TRASH = 10000
PER_TILE = NPAD // NS
IDXW = 4


WB = 4


def _sc_spmm_call(table, gidx, sidx, zeros_rows, cpw):
    mesh = plsc.VectorSubcoreMesh(core_axis_name="c", subcore_axis_name="s")
    wa = 2 * (cpw // IDXW) - WB

    out_type = [jax.ShapeDtypeStruct((NC, NPAD, C), jnp.float32)]
    scratch = [
        pltpu.VMEM((IDXW, CHUNK), jnp.int32),
        pltpu.VMEM((IDXW, CHUNK), jnp.int32),
        pltpu.VMEM((IDXW, CHUNK), jnp.int32),
        pltpu.VMEM((IDXW, CHUNK), jnp.int32),
        pltpu.VMEM((CHUNK, C), jnp.float32),
        pltpu.VMEM((CHUNK, C), jnp.float32),
        pltpu.VMEM_SHARED((NPAD, C), jnp.float32),
        pltpu.SemaphoreType.DMA,
        pltpu.SemaphoreType.DMA,
        pltpu.SemaphoreType.DMA,
        pltpu.SemaphoreType.DMA,
        pltpu.SemaphoreType.DMA,
        pltpu.SemaphoreType.DMA,
    ]

    def body(table_h, gidx_h, sidx_h, zrows_h, out_h,
             gi0, gi1, si0, si1, r0, r1, acc_sh,
             ga0, ga1, sc0, sc1, ix0, ix1):
        c = lax.axis_index("c")
        s = lax.axis_index("s")
        row0 = s * PER_TILE
        pltpu.sync_copy(zrows_h, acc_sh.at[pl.ds(row0, PER_TILE)])
        plsc.subcore_barrier()
        bufs = ((gi0, gi1), (si0, si1), (r0, r1),
                (ga0, ga1), (sc0, sc1), (ix0, ix1))

        @pl.when(c == 0)
        def _():
            _emit_spmm_pipeline(table_h, gidx_h, sidx_h, acc_sh, *bufs,
                                s * wa, wa)

        @pl.when(c == 1)
        def _():
            _emit_spmm_pipeline(table_h, gidx_h, sidx_h, acc_sh, *bufs,
                                NS * wa + s * WB, WB)

        plsc.subcore_barrier()
        pltpu.sync_copy(acc_sh.at[pl.ds(row0, PER_TILE)],
                        out_h.at[c, pl.ds(row0, PER_TILE)])

    fn = pl.kernel(body, out_type=out_type, mesh=mesh, scratch_types=scratch,
                   name="sc_spmm")
    return fn(table, gidx, sidx, zeros_rows)


def _emit_spmm_pipeline(table_h, gidx_h, sidx_h, acc_sh, gi, si, rows,
                        ga, sc, ix, wbase, wpw):
    def idx_load(w, p):
        pltpu.make_async_copy(gidx_h.at[wbase + w], gi[p], ix[p]).start()
        pltpu.make_async_copy(sidx_h.at[wbase + w], si[p], ix[p]).start()

    def idx_wait(p):
        pltpu.make_async_copy(gidx_h.at[wbase], gi[p], ix[p]).wait()
        pltpu.make_async_copy(sidx_h.at[wbase], si[p], ix[p]).wait()

    def gather_start(p, k, b):
        pltpu.make_async_copy(
            table_h.at[gi[p].at[k]], rows[b], ga[b]).start()

    def gather_wait(b):
        pltpu.make_async_copy(
            table_h.at[gi[0].at[0]], rows[b], ga[b]).wait()

    def scatter_start(p, k, b):
        pltpu.make_async_copy(
            rows[b], acc_sh.at[si[p].at[k]], sc[b]).start(add=True)

    def scatter_wait(b):
        pltpu.make_async_copy(
            rows[b], acc_sh.at[si[0].at[0]], sc[b]).wait()

    def emit_window(w, p, first=False, last=False):
        for k in range(IDXW):
            b = k % 2
            gather_wait(b)
            if not (first and k == 0):
                scatter_wait(1 - b)
            if k == 0 and not first and not last:
                idx_load(w + 1, 1 - p)
            if k < IDXW - 1:
                gather_start(p, k + 1, 1 - b)
            elif not last:
                if not first:
                    idx_wait(1 - p)
                gather_start(1 - p, 0, 0)
            scatter_start(p, k, b)

    pltpu.sync_copy(gidx_h.at[wbase], gi[0])
    pltpu.sync_copy(sidx_h.at[wbase], si[0])
    pltpu.sync_copy(gidx_h.at[wbase + 1], gi[1])
    pltpu.sync_copy(sidx_h.at[wbase + 1], si[1])
    gather_start(0, 0, 0)
    emit_window(0, 0, first=True)
    emit_window(1, 1)

    def pair(g, carry):
        emit_window(2 * g, 0)
        emit_window(2 * g + 1, 1)
        return carry

    lax.fori_loop(1, wpw // 2 - 1, pair, 0)
    emit_window(wpw - 2, 0)
    emit_window(wpw - 1, 1, last=True)
    scatter_wait(1)


def _sc_spmm_deg_call(table, gidx, sidx, zeros_rows, ones_rows, cpw):
    mesh = plsc.VectorSubcoreMesh(core_axis_name="c", subcore_axis_name="s")
    wpw2 = 2 * (cpw // IDXW)

    out_type = [jax.ShapeDtypeStruct((NPAD, C), jnp.float32),
                jax.ShapeDtypeStruct((NPAD, C), jnp.float32),
                jax.ShapeDtypeStruct((NPAD, C), jnp.float32)]
    scratch = [
        pltpu.VMEM((IDXW, CHUNK), jnp.int32),
        pltpu.VMEM((IDXW, CHUNK), jnp.int32),
        pltpu.VMEM((IDXW, CHUNK), jnp.int32),
        pltpu.VMEM((IDXW, CHUNK), jnp.int32),
        pltpu.VMEM((CHUNK, C), jnp.float32),
        pltpu.VMEM((CHUNK, C), jnp.float32),
        pltpu.VMEM_SHARED((NPAD, C), jnp.float32),
        pltpu.SemaphoreType.DMA,
        pltpu.SemaphoreType.DMA,
        pltpu.SemaphoreType.DMA,
        pltpu.SemaphoreType.DMA,
        pltpu.SemaphoreType.DMA,
        pltpu.SemaphoreType.DMA,
    ]

    def body(table_h, gidx_h, sidx_h, zrows_h, ones_h,
             out_h, dege_h, degv_h,
             gi0, gi1, si0, si1, r0, r1, acc_sh,
             ga0, ga1, sc0, sc1, ix0, ix1):
        c = lax.axis_index("c")
        s = lax.axis_index("s")
        wbase = s * wpw2
        row0 = s * PER_TILE
        bufs = ((gi0, gi1), (si0, si1), (r0, r1),
                (ga0, ga1), (sc0, sc1), (ix0, ix1))

        @pl.when(c == 0)
        def _spmm():
            pltpu.sync_copy(zrows_h, acc_sh.at[pl.ds(row0, PER_TILE)])
            plsc.subcore_barrier()
            _emit_spmm_pipeline(table_h, gidx_h, sidx_h, acc_sh, *bufs,
                                s * wpw2, wpw2)
            plsc.subcore_barrier()
            pltpu.sync_copy(acc_sh.at[pl.ds(row0, PER_TILE)],
                            out_h.at[pl.ds(row0, PER_TILE)])

        @pl.when(c == 1)
        def _deg():
            pltpu.sync_copy(ones_h, r0)
            for idx_h, i_v, dout_h in ((sidx_h, si0, dege_h),
                                       (gidx_h, gi0, degv_h)):
                pltpu.sync_copy(zrows_h, acc_sh.at[pl.ds(row0, PER_TILE)])
                plsc.subcore_barrier()

                def outer(g, carry):
                    pltpu.sync_copy(idx_h.at[wbase + g], i_v)
                    for k in range(IDXW):
                        pltpu.sync_copy(r0, acc_sh.at[i_v.at[k]], add=True)
                    return carry

                lax.fori_loop(0, wpw2, outer, 0)
                plsc.subcore_barrier()
                pltpu.sync_copy(acc_sh.at[pl.ds(row0, PER_TILE)],
                                dout_h.at[pl.ds(row0, PER_TILE)])
                plsc.subcore_barrier()

    fn = pl.kernel(body, out_type=out_type, mesh=mesh, scratch_types=scratch,
                   name="sc_spmm_deg")
    return fn(table, gidx, sidx, zeros_rows, ones_rows)


BM = 1264


def _mm(x, w, b):
    def mm_body(x_ref, w_ref, b_ref, o_ref):
        o_ref[...] = (jnp.dot(x_ref[...], w_ref[...],
                              preferred_element_type=jnp.float32)
                      + b_ref[...])

    m = x.shape[0]
    return pl.pallas_call(
        mm_body,
        grid=(m // BM,),
        in_specs=[pl.BlockSpec((BM, C), lambda i: (i, 0)),
                  pl.BlockSpec((C, C), lambda i: (0, 0)),
                  pl.BlockSpec((1, C), lambda i: (0, 0))],
        out_specs=pl.BlockSpec((BM, C), lambda i: (i, 0)),
        out_shape=jax.ShapeDtypeStruct((m, C), jnp.float32),
    )(x, w, b.reshape(1, C))


def _combine(p0, p1, d, w, b, a11, prelu, mm):
    def comb_body(*refs):
        i = 0
        p0_ref = refs[i]; i += 1
        if p1 is not None:
            p1_ref = refs[i]; i += 1
        d_ref = refs[i]; i += 1
        a_ref = refs[i]; i += 1
        if mm:
            w_ref = refs[i]; b_ref = refs[i + 1]
            v_ref = refs[i + 2]; o_ref = refs[i + 3]
        else:
            v_ref = refs[i]
        ssum = p0_ref[...]
        if p1 is not None:
            ssum = ssum + p1_ref[...]
        deg = jnp.maximum(d_ref[...], 1.0)
        v = ssum / deg
        if prelu:
            v = jnp.where(v >= 0, v, a_ref[0, 0] * v)
        v_ref[...] = v
        if mm:
            o_ref[...] = (jnp.dot(v, w_ref[...],
                                  preferred_element_type=jnp.float32)
                          + b_ref[...])

    m = p0.shape[0]
    in_specs = [pl.BlockSpec((BM, C), lambda i: (i, 0))]
    args = [p0]
    if p1 is not None:
        in_specs.append(pl.BlockSpec((BM, C), lambda i: (i, 0)))
        args.append(p1)
    in_specs += [pl.BlockSpec((BM, 1), lambda i: (i, 0)),
                 pl.BlockSpec((1, 1), lambda i: (0, 0))]
    args += [d, a11]
    out_shape = [jax.ShapeDtypeStruct((m, C), jnp.float32)]
    out_specs = [pl.BlockSpec((BM, C), lambda i: (i, 0))]
    if mm:
        in_specs += [pl.BlockSpec((C, C), lambda i: (0, 0)),
                     pl.BlockSpec((1, C), lambda i: (0, 0))]
        args += [w, b.reshape(1, C)]
        out_shape += [jax.ShapeDtypeStruct((m, C), jnp.float32)]
        out_specs += [pl.BlockSpec((BM, C), lambda i: (i, 0))]

    res = pl.pallas_call(
        comb_body,
        grid=(m // BM,),
        in_specs=in_specs,
        out_specs=out_specs,
        out_shape=out_shape,
    )(*args)
    return res if mm else res[0]


def kernel(x, hyperedge_index, num_nodes, num_edges,
           w1_0, b1_0, w2_0, b2_0, w1_1, b1_1, w2_1, b2_1, prelu_a):
    n = x.shape[0]
    nnz = hyperedge_index.shape[1]
    cpw = -(-nnz // (NW * CHUNK))
    cpw = -(-cpw // (2 * IDXW)) * (2 * IDXW)
    nnz_pad = NW * cpw * CHUNK

    node_idx = hyperedge_index[0].astype(jnp.int32)
    edge_idx = hyperedge_index[1].astype(jnp.int32)
    pad = jnp.full((nnz_pad - nnz,), TRASH, jnp.int32)
    nidx = jnp.concatenate([node_idx, pad]).reshape(-1, IDXW, CHUNK)
    eidx = jnp.concatenate([edge_idx, pad]).reshape(-1, IDXW, CHUNK)

    xp = jnp.concatenate(
        [x.astype(jnp.float32), jnp.zeros((NPAD - n, C), jnp.float32)])
    zeros_rows = jnp.zeros((PER_TILE, C), jnp.float32)
    a11 = prelu_a.reshape(1, 1).astype(jnp.float32)

    spmm = functools.partial(_sc_spmm_call, zeros_rows=zeros_rows, cpw=cpw)

    ones_rows = jnp.ones((CHUNK, C), jnp.float32)
    x1 = _mm(xp, w1_0, b1_0)
    ep, dege, degv = _sc_spmm_deg_call(x1, nidx, eidx, zeros_rows,
                                       ones_rows, cpw)
    de = dege[:, 0:1]
    dv = degv[:, 0:1]
    _, m0 = _combine(ep, None, de, w2_0, b2_0, a11, prelu=False, mm=True)
    (xsp,) = spmm(m0, eidx, nidx)
    _, x1l1 = _combine(xsp[0], xsp[1], dv, w1_1, b1_1, a11,
                       prelu=True, mm=True)
    (ep1,) = spmm(x1l1, nidx, eidx)
    e1, m1 = _combine(ep1[0], ep1[1], de, w2_1, b2_1, a11,
                      prelu=False, mm=True)
    (xsp1,) = spmm(m1, eidx, nidx)
    xf = _combine(xsp1[0], xsp1[1], dv, None, None, a11,
                  prelu=True, mm=False)
    return (xf[:n], e1[:n])
